# baseline (speedup 1.0000x reference)
import functools

import jax
import jax.numpy as jnp
from jax import lax
from jax.experimental import pallas as pl
from jax.experimental.pallas import tpu as pltpu
from jax.experimental.pallas import tpu_sc as plsc

N = 100000
ATT = 16
EMB = 128
OUT = 3 * ATT + EMB

NC = 2
NS = 16
NW = NC * NS

C = 80
NG = N // C


def _body(age_hbm, gen_hbm, city_hbm, emb_hbm, wa_hbm, wg_hbm, wc_hbm,
          out_hbm, ia, ig, ic, ra, rg, rc, re, sem):
  wid = lax.axis_index("s") * NC + lax.axis_index("c")

  @pl.loop(wid, NG, step=NW)
  def chunk(g):
    base = g * C
    pltpu.sync_copy(age_hbm.at[pl.ds(base, C)], ia)
    pltpu.sync_copy(gen_hbm.at[pl.ds(base, C)], ig)
    pltpu.sync_copy(city_hbm.at[pl.ds(base, C)], ic)
    pltpu.async_copy(wa_hbm.at[ia], ra, sem).wait()
    pltpu.async_copy(wg_hbm.at[ig], rg, sem).wait()
    pltpu.async_copy(wc_hbm.at[ic], rc, sem).wait()
    pltpu.sync_copy(emb_hbm.at[pl.ds(base, C)], re)
    pltpu.sync_copy(ra, out_hbm.at[pl.ds(base, C), pl.ds(0, ATT)])
    pltpu.sync_copy(rg, out_hbm.at[pl.ds(base, C), pl.ds(ATT, ATT)])
    pltpu.sync_copy(rc, out_hbm.at[pl.ds(base, C), pl.ds(2 * ATT, ATT)])
    pltpu.sync_copy(re, out_hbm.at[pl.ds(base, C), pl.ds(3 * ATT, EMB)])


@jax.jit
def _run(age_idx, gender_idx, city_idx, emb_feat, W_age, W_gender, W_city):
  mesh = plsc.VectorSubcoreMesh(core_axis_name="c", subcore_axis_name="s")
  f = pl.kernel(
      _body,
      out_type=jax.ShapeDtypeStruct((N, OUT), jnp.float32),
      mesh=mesh,
      compiler_params=pltpu.CompilerParams(use_tc_tiling_on_sc=False),
      scratch_types=[
          pltpu.VMEM((C,), jnp.int32),
          pltpu.VMEM((C,), jnp.int32),
          pltpu.VMEM((C,), jnp.int32),
          pltpu.VMEM((C, ATT), jnp.float32),
          pltpu.VMEM((C, ATT), jnp.float32),
          pltpu.VMEM((C, ATT), jnp.float32),
          pltpu.VMEM((C, EMB), jnp.float32),
          pltpu.SemaphoreType.DMA,
      ],
  )
  return f(age_idx, gender_idx, city_idx, emb_feat, W_age, W_gender, W_city)


def kernel(age_idx, gender_idx, city_idx, emb_feat, W_age, W_gender, W_city):
  return _run(
      age_idx.astype(jnp.int32),
      gender_idx.astype(jnp.int32),
      city_idx.astype(jnp.int32),
      emb_feat, W_age, W_gender, W_city)


# speedup vs baseline: 1.0069x; 1.0069x over previous
import jax
import jax.numpy as jnp
from jax import lax
from jax.experimental import pallas as pl
from jax.experimental.pallas import tpu as pltpu
from jax.experimental.pallas import tpu_sc as plsc

N = 100000
ATT = 16
EMB = 128
OUT = 3 * ATT + EMB

NC = 2
NS = 16
NW = NC * NS

C = 128
NG = N // C
TAIL = N - NG * C

NSLOT = 4


def _body(age_hbm, gen_hbm, city_hbm, emb_hbm, wa_hbm, wg_hbm, wc_hbm,
          out_hbm, ia, ig, ic, ra, rg, rc, re, sa, sb, sc_):
  wid = lax.axis_index("s") * NC + lax.axis_index("c")

  def issue_idx(base, n, s):
    pltpu.async_copy(age_hbm.at[pl.ds(base, n)], ia.at[s, pl.ds(0, n)], sa)
    pltpu.async_copy(gen_hbm.at[pl.ds(base, n)], ig.at[s, pl.ds(0, n)], sa)
    pltpu.async_copy(city_hbm.at[pl.ds(base, n)], ic.at[s, pl.ds(0, n)], sa)

  def wait_idx(n, s):
    pltpu.make_async_copy(age_hbm.at[pl.ds(0, n)], ia.at[s, pl.ds(0, n)], sa).wait()
    pltpu.make_async_copy(gen_hbm.at[pl.ds(0, n)], ig.at[s, pl.ds(0, n)], sa).wait()
    pltpu.make_async_copy(city_hbm.at[pl.ds(0, n)], ic.at[s, pl.ds(0, n)], sa).wait()

  def issue_loads(base, n, s):
    pltpu.async_copy(wa_hbm.at[ia.at[s, pl.ds(0, n)]], ra.at[s, pl.ds(0, n)], sb)
    pltpu.async_copy(wg_hbm.at[ig.at[s, pl.ds(0, n)]], rg.at[s, pl.ds(0, n)], sb)
    pltpu.async_copy(wc_hbm.at[ic.at[s, pl.ds(0, n)]], rc.at[s, pl.ds(0, n)], sb)
    pltpu.async_copy(emb_hbm.at[pl.ds(base, n)], re.at[s, pl.ds(0, n)], sb)

  def wait_loads(n, s):
    pltpu.make_async_copy(wa_hbm.at[ia.at[s, pl.ds(0, n)]], ra.at[s, pl.ds(0, n)], sb).wait()
    pltpu.make_async_copy(wg_hbm.at[ig.at[s, pl.ds(0, n)]], rg.at[s, pl.ds(0, n)], sb).wait()
    pltpu.make_async_copy(wc_hbm.at[ic.at[s, pl.ds(0, n)]], rc.at[s, pl.ds(0, n)], sb).wait()
    pltpu.make_async_copy(emb_hbm.at[pl.ds(0, n)], re.at[s, pl.ds(0, n)], sb).wait()

  def issue_writes(base, n, s):
    pltpu.async_copy(ra.at[s, pl.ds(0, n)], out_hbm.at[pl.ds(base, n), pl.ds(0, ATT)], sc_)
    pltpu.async_copy(rg.at[s, pl.ds(0, n)], out_hbm.at[pl.ds(base, n), pl.ds(ATT, ATT)], sc_)
    pltpu.async_copy(rc.at[s, pl.ds(0, n)], out_hbm.at[pl.ds(base, n), pl.ds(2 * ATT, ATT)], sc_)
    pltpu.async_copy(re.at[s, pl.ds(0, n)], out_hbm.at[pl.ds(base, n), pl.ds(3 * ATT, EMB)], sc_)

  def wait_writes(n, s):
    pltpu.make_async_copy(ra.at[s, pl.ds(0, n)], out_hbm.at[pl.ds(0, n), pl.ds(0, ATT)], sc_).wait()
    pltpu.make_async_copy(rg.at[s, pl.ds(0, n)], out_hbm.at[pl.ds(0, n), pl.ds(ATT, ATT)], sc_).wait()
    pltpu.make_async_copy(rc.at[s, pl.ds(0, n)], out_hbm.at[pl.ds(0, n), pl.ds(2 * ATT, ATT)], sc_).wait()
    pltpu.make_async_copy(re.at[s, pl.ds(0, n)], out_hbm.at[pl.ds(0, n), pl.ds(3 * ATT, EMB)], sc_).wait()

  @pl.loop(wid, NG, step=NW * NSLOT)
  def block(g):
    for s in range(NSLOT):
      @pl.when(g + s * NW < NG)
      def _():
        issue_idx((g + s * NW) * C, C, s)
    for s in range(NSLOT):
      @pl.when(g + s * NW < NG)
      def _():
        wait_idx(C, s)
        issue_loads((g + s * NW) * C, C, s)
    for s in range(NSLOT):
      @pl.when(g + s * NW < NG)
      def _():
        wait_loads(C, s)
        issue_writes((g + s * NW) * C, C, s)
    for s in range(NSLOT):
      @pl.when(g + s * NW < NG)
      def _():
        wait_writes(C, s)

  if TAIL:
    @pl.when(wid == NW - 1)
    def _tail():
      base = NG * C
      issue_idx(base, TAIL, 0)
      wait_idx(TAIL, 0)
      issue_loads(base, TAIL, 0)
      wait_loads(TAIL, 0)
      issue_writes(base, TAIL, 0)
      wait_writes(TAIL, 0)


@jax.jit
def _run(age_idx, gender_idx, city_idx, emb_feat, W_age, W_gender, W_city):
  mesh = plsc.VectorSubcoreMesh(core_axis_name="c", subcore_axis_name="s")
  f = pl.kernel(
      _body,
      out_type=jax.ShapeDtypeStruct((N, OUT), jnp.float32),
      mesh=mesh,
      compiler_params=pltpu.CompilerParams(use_tc_tiling_on_sc=False),
      scratch_types=[
          pltpu.VMEM((NSLOT, C), jnp.int32),
          pltpu.VMEM((NSLOT, C), jnp.int32),
          pltpu.VMEM((NSLOT, C), jnp.int32),
          pltpu.VMEM((NSLOT, C, ATT), jnp.float32),
          pltpu.VMEM((NSLOT, C, ATT), jnp.float32),
          pltpu.VMEM((NSLOT, C, ATT), jnp.float32),
          pltpu.VMEM((NSLOT, C, EMB), jnp.float32),
          pltpu.SemaphoreType.DMA,
          pltpu.SemaphoreType.DMA,
          pltpu.SemaphoreType.DMA,
      ],
  )
  return f(age_idx, gender_idx, city_idx, emb_feat, W_age, W_gender, W_city)


def kernel(age_idx, gender_idx, city_idx, emb_feat, W_age, W_gender, W_city):
  return _run(
      age_idx.astype(jnp.int32),
      gender_idx.astype(jnp.int32),
      city_idx.astype(jnp.int32),
      emb_feat, W_age, W_gender, W_city)


# speedup vs baseline: 1.0111x; 1.0041x over previous
import jax
import jax.numpy as jnp
from jax import lax
from jax.experimental import pallas as pl
from jax.experimental.pallas import tpu as pltpu
from jax.experimental.pallas import tpu_sc as plsc

N = 100000
ATT = 16
EMB = 128
OUT = 3 * ATT + EMB

NC = 2
NS = 16
NW = NC * NS

C = 128
NG = N // C
TAIL = N - NG * C

NSLOT = 4


def _body(age_hbm, gen_hbm, city_hbm, emb_hbm, wa_hbm, wg_hbm, wc_hbm,
          out_hbm, ia, ig, ic, ra, rg, rc, ob, sa, sb, sc_):
  wid = lax.axis_index("s") * NC + lax.axis_index("c")

  def issue_idx(base, n, s):
    pltpu.async_copy(age_hbm.at[pl.ds(base, n)], ia.at[s, pl.ds(0, n)], sa)
    pltpu.async_copy(gen_hbm.at[pl.ds(base, n)], ig.at[s, pl.ds(0, n)], sa)
    pltpu.async_copy(city_hbm.at[pl.ds(base, n)], ic.at[s, pl.ds(0, n)], sa)

  def wait_idx(n, s):
    pltpu.make_async_copy(age_hbm.at[pl.ds(0, n)], ia.at[s, pl.ds(0, n)], sa).wait()
    pltpu.make_async_copy(gen_hbm.at[pl.ds(0, n)], ig.at[s, pl.ds(0, n)], sa).wait()
    pltpu.make_async_copy(city_hbm.at[pl.ds(0, n)], ic.at[s, pl.ds(0, n)], sa).wait()

  def issue_loads(base, n, s):
    pltpu.async_copy(wa_hbm.at[ia.at[s, pl.ds(0, n)]], ra.at[s, pl.ds(0, n)], sb)
    pltpu.async_copy(wg_hbm.at[ig.at[s, pl.ds(0, n)]], rg.at[s, pl.ds(0, n)], sb)
    pltpu.async_copy(wc_hbm.at[ic.at[s, pl.ds(0, n)]], rc.at[s, pl.ds(0, n)], sb)
    pltpu.async_copy(emb_hbm.at[pl.ds(base, n)],
                     ob.at[s, pl.ds(0, n), pl.ds(3 * ATT, EMB)], sb)

  def wait_loads(n, s):
    pltpu.make_async_copy(wa_hbm.at[ia.at[s, pl.ds(0, n)]], ra.at[s, pl.ds(0, n)], sb).wait()
    pltpu.make_async_copy(wg_hbm.at[ig.at[s, pl.ds(0, n)]], rg.at[s, pl.ds(0, n)], sb).wait()
    pltpu.make_async_copy(wc_hbm.at[ic.at[s, pl.ds(0, n)]], rc.at[s, pl.ds(0, n)], sb).wait()
    pltpu.make_async_copy(emb_hbm.at[pl.ds(0, n)],
                          ob.at[s, pl.ds(0, n), pl.ds(3 * ATT, EMB)], sb).wait()

  def merge_atts(n, s):
    @pl.loop(0, n, unroll=4)
    def row(r):
      ob[s, r, pl.ds(0, ATT)] = ra[s, r, :]
      ob[s, r, pl.ds(ATT, ATT)] = rg[s, r, :]
      ob[s, r, pl.ds(2 * ATT, ATT)] = rc[s, r, :]

  def issue_write(base, n, s):
    pltpu.async_copy(ob.at[s, pl.ds(0, n)], out_hbm.at[pl.ds(base, n)], sc_)

  def wait_write(n, s):
    pltpu.make_async_copy(ob.at[s, pl.ds(0, n)], out_hbm.at[pl.ds(0, n)], sc_).wait()

  @pl.loop(wid, NG, step=NW * NSLOT)
  def block(g):
    for s in range(NSLOT):
      @pl.when(g + s * NW < NG)
      def _():
        issue_idx((g + s * NW) * C, C, s)
    for s in range(NSLOT):
      @pl.when(g + s * NW < NG)
      def _():
        wait_idx(C, s)
        issue_loads((g + s * NW) * C, C, s)
    for s in range(NSLOT):
      @pl.when(g + s * NW < NG)
      def _():
        wait_loads(C, s)
        merge_atts(C, s)
        issue_write((g + s * NW) * C, C, s)
    for s in range(NSLOT):
      @pl.when(g + s * NW < NG)
      def _():
        wait_write(C, s)

  if TAIL:
    @pl.when(wid == NW - 1)
    def _tail():
      base = NG * C
      issue_idx(base, TAIL, 0)
      wait_idx(TAIL, 0)
      issue_loads(base, TAIL, 0)
      wait_loads(TAIL, 0)
      merge_atts(TAIL, 0)
      issue_write(base, TAIL, 0)
      wait_write(TAIL, 0)


@jax.jit
def _run(age_idx, gender_idx, city_idx, emb_feat, W_age, W_gender, W_city):
  mesh = plsc.VectorSubcoreMesh(core_axis_name="c", subcore_axis_name="s")
  f = pl.kernel(
      _body,
      out_type=jax.ShapeDtypeStruct((N, OUT), jnp.float32),
      mesh=mesh,
      compiler_params=pltpu.CompilerParams(use_tc_tiling_on_sc=False),
      scratch_types=[
          pltpu.VMEM((NSLOT, C), jnp.int32),
          pltpu.VMEM((NSLOT, C), jnp.int32),
          pltpu.VMEM((NSLOT, C), jnp.int32),
          pltpu.VMEM((NSLOT, C, ATT), jnp.float32),
          pltpu.VMEM((NSLOT, C, ATT), jnp.float32),
          pltpu.VMEM((NSLOT, C, ATT), jnp.float32),
          pltpu.VMEM((NSLOT, C, OUT), jnp.float32),
          pltpu.SemaphoreType.DMA,
          pltpu.SemaphoreType.DMA,
          pltpu.SemaphoreType.DMA,
      ],
  )
  return f(age_idx, gender_idx, city_idx, emb_feat, W_age, W_gender, W_city)


def kernel(age_idx, gender_idx, city_idx, emb_feat, W_age, W_gender, W_city):
  return _run(
      age_idx.astype(jnp.int32),
      gender_idx.astype(jnp.int32),
      city_idx.astype(jnp.int32),
      emb_feat, W_age, W_gender, W_city)


# speedup vs baseline: 1.5381x; 1.5213x over previous
import functools

import jax
import jax.numpy as jnp
from jax import lax
from jax.experimental import pallas as pl
from jax.experimental.pallas import tpu as pltpu
from jax.experimental.pallas import tpu_sc as plsc

N = 100000
ATT = 16
EMB = 128
OUT = 3 * ATT + EMB
APAD = 128

NC = 2
NS = 16
NW = NC * NS

C = 128
NG = N // C
TAIL = N - NG * C

NSLOT = 4


def _sc_body(age_hbm, gen_hbm, city_hbm, wa_hbm, wg_hbm, wc_hbm,
             att_hbm, ia, ig, ic, ra, rg, rc, sa, sb, sc_):
  wid = lax.axis_index("s") * NC + lax.axis_index("c")

  def issue_idx(base, n, s):
    pltpu.async_copy(age_hbm.at[pl.ds(base, n)], ia.at[s, pl.ds(0, n)], sa)
    pltpu.async_copy(gen_hbm.at[pl.ds(base, n)], ig.at[s, pl.ds(0, n)], sa)
    pltpu.async_copy(city_hbm.at[pl.ds(base, n)], ic.at[s, pl.ds(0, n)], sa)

  def wait_idx(n, s):
    pltpu.make_async_copy(age_hbm.at[pl.ds(0, n)], ia.at[s, pl.ds(0, n)], sa).wait()
    pltpu.make_async_copy(gen_hbm.at[pl.ds(0, n)], ig.at[s, pl.ds(0, n)], sa).wait()
    pltpu.make_async_copy(city_hbm.at[pl.ds(0, n)], ic.at[s, pl.ds(0, n)], sa).wait()

  def issue_gathers(n, s):
    pltpu.async_copy(wa_hbm.at[ia.at[s, pl.ds(0, n)]], ra.at[s, pl.ds(0, n)], sb)
    pltpu.async_copy(wg_hbm.at[ig.at[s, pl.ds(0, n)]], rg.at[s, pl.ds(0, n)], sb)
    pltpu.async_copy(wc_hbm.at[ic.at[s, pl.ds(0, n)]], rc.at[s, pl.ds(0, n)], sb)

  def wait_gathers(n, s):
    pltpu.make_async_copy(wa_hbm.at[ia.at[s, pl.ds(0, n)]], ra.at[s, pl.ds(0, n)], sb).wait()
    pltpu.make_async_copy(wg_hbm.at[ig.at[s, pl.ds(0, n)]], rg.at[s, pl.ds(0, n)], sb).wait()
    pltpu.make_async_copy(wc_hbm.at[ic.at[s, pl.ds(0, n)]], rc.at[s, pl.ds(0, n)], sb).wait()

  def issue_writes(base, n, s):
    pltpu.async_copy(ra.at[s, pl.ds(0, n)],
                     att_hbm.at[pl.ds(base, n), pl.ds(0, ATT)], sc_)
    pltpu.async_copy(rg.at[s, pl.ds(0, n)],
                     att_hbm.at[pl.ds(base, n), pl.ds(ATT, ATT)], sc_)
    pltpu.async_copy(rc.at[s, pl.ds(0, n)],
                     att_hbm.at[pl.ds(base, n), pl.ds(2 * ATT, ATT)], sc_)

  def wait_writes(n, s):
    pltpu.make_async_copy(ra.at[s, pl.ds(0, n)],
                          att_hbm.at[pl.ds(0, n), pl.ds(0, ATT)], sc_).wait()
    pltpu.make_async_copy(rg.at[s, pl.ds(0, n)],
                          att_hbm.at[pl.ds(0, n), pl.ds(ATT, ATT)], sc_).wait()
    pltpu.make_async_copy(rc.at[s, pl.ds(0, n)],
                          att_hbm.at[pl.ds(0, n), pl.ds(2 * ATT, ATT)], sc_).wait()

  @pl.loop(wid, NG, step=NW * NSLOT)
  def block(g):
    for s in range(NSLOT):
      @pl.when(g + s * NW < NG)
      def _():
        issue_idx((g + s * NW) * C, C, s)
    for s in range(NSLOT):
      @pl.when(g + s * NW < NG)
      def _():
        wait_idx(C, s)
        issue_gathers(C, s)
    for s in range(NSLOT):
      @pl.when(g + s * NW < NG)
      def _():
        wait_gathers(C, s)
        issue_writes((g + s * NW) * C, C, s)
    for s in range(NSLOT):
      @pl.when(g + s * NW < NG)
      def _():
        wait_writes(C, s)

  if TAIL:
    @pl.when(wid == NW - 1)
    def _tail():
      base = NG * C
      issue_idx(base, TAIL, 0)
      wait_idx(TAIL, 0)
      issue_gathers(TAIL, 0)
      wait_gathers(TAIL, 0)
      issue_writes(base, TAIL, 0)
      wait_writes(TAIL, 0)


def _sc_gather(age_idx, gender_idx, city_idx, W_age, W_gender, W_city):
  mesh = plsc.VectorSubcoreMesh(core_axis_name="c", subcore_axis_name="s")
  f = pl.kernel(
      _sc_body,
      out_type=jax.ShapeDtypeStruct((N, APAD), jnp.float32),
      mesh=mesh,
      compiler_params=pltpu.CompilerParams(use_tc_tiling_on_sc=False),
      scratch_types=[
          pltpu.VMEM((NSLOT, C), jnp.int32),
          pltpu.VMEM((NSLOT, C), jnp.int32),
          pltpu.VMEM((NSLOT, C), jnp.int32),
          pltpu.VMEM((NSLOT, C, ATT), jnp.float32),
          pltpu.VMEM((NSLOT, C, ATT), jnp.float32),
          pltpu.VMEM((NSLOT, C, ATT), jnp.float32),
          pltpu.SemaphoreType.DMA,
          pltpu.SemaphoreType.DMA,
          pltpu.SemaphoreType.DMA,
      ],
  )
  return f(age_idx, gender_idx, city_idx, W_age, W_gender, W_city)


B = 1000


def _tc_body(att_ref, emb_ref, out_ref):
  out_ref[:, :] = jnp.concatenate(
      [att_ref[:, : 3 * ATT], emb_ref[:, :]], axis=1)


def _tc_concat(att, emb_feat):
  return pl.pallas_call(
      _tc_body,
      grid=(N // B,),
      in_specs=[
          pl.BlockSpec((B, APAD), lambda i: (i, 0)),
          pl.BlockSpec((B, EMB), lambda i: (i, 0)),
      ],
      out_specs=pl.BlockSpec((B, OUT), lambda i: (i, 0)),
      out_shape=jax.ShapeDtypeStruct((N, OUT), jnp.float32),
  )(att, emb_feat)


@jax.jit
def _run(age_idx, gender_idx, city_idx, emb_feat, W_age, W_gender, W_city):
  att = _sc_gather(age_idx, gender_idx, city_idx, W_age, W_gender, W_city)
  return _tc_concat(att, emb_feat)


def kernel(age_idx, gender_idx, city_idx, emb_feat, W_age, W_gender, W_city):
  return _run(
      age_idx.astype(jnp.int32),
      gender_idx.astype(jnp.int32),
      city_idx.astype(jnp.int32),
      emb_feat, W_age, W_gender, W_city)
